# baseline (speedup 1.0000x reference)
import functools

import jax
import jax.numpy as jnp
from jax import lax
from jax.experimental import pallas as pl
from jax.experimental.pallas import tpu as pltpu
from jax.experimental.pallas import tpu_sc as plsc

N = 10000
E = 320000
C = 128
H = 128

NC = 2
NS = 16
NT = NC * NS
K = 128
NCH = 79
EPAD = NT * NCH * K
NPAD = 10240
PAD_DST = 10016
DEGW = 128
RPT = NPAD // NS
NBUF = 2
WCH = 16
NW = NCH // WCH
DDEPTH = 8

_ROWS = 1000

_mesh = plsc.VectorSubcoreMesh(core_axis_name="c", subcore_axis_name="s")


def _deg_body(dst_hbm, const_hbm, out_hbm, dstv, onesv, zv, deg_sh, zsem,
              dsem):
    c = lax.axis_index("c")
    s = lax.axis_index("s")
    wid = c * NS + s

    pltpu.sync_copy(const_hbm.at[pl.ds(0, K)], onesv)
    pltpu.sync_copy(const_hbm.at[pl.ds(K, 64)], zv)

    nz = RPT // 64
    for t in range(nz):
        pltpu.async_copy(zv, deg_sh.at[pl.ds(s * RPT + t * 64, 64)], zsem)
    pltpu.sync_copy(dst_hbm.at[wid], dstv)
    for t in range(nz):
        pltpu.make_async_copy(zv, deg_sh.at[pl.ds(s * RPT + t * 64, 64)],
                              zsem).wait()
    plsc.subcore_barrier()

    for b in range(DDEPTH):
        pltpu.async_copy(onesv, deg_sh.at[dstv.at[b]], dsem, add=True)

    def chunk(j, _):
        pltpu.make_async_copy(onesv, deg_sh.at[dstv.at[j]], dsem).wait()
        pltpu.async_copy(onesv, deg_sh.at[dstv.at[j + DDEPTH]], dsem,
                         add=True)
        return 0

    lax.fori_loop(0, NCH - DDEPTH, chunk, 0)

    def drain(j, _):
        pltpu.make_async_copy(onesv, deg_sh.at[dstv.at[j]], dsem).wait()
        return 0

    lax.fori_loop(0, DDEPTH, drain, 0)
    plsc.subcore_barrier()
    pltpu.sync_copy(deg_sh.at[pl.ds(s * RPT, RPT)],
                    out_hbm.at[c, pl.ds(s * RPT, RPT)])


_sc_deg = functools.partial(
    pl.kernel,
    _deg_body,
    out_type=jax.ShapeDtypeStruct((NC, NPAD, DEGW), jnp.float32),
    mesh=_mesh,
    scratch_types=[
        pltpu.VMEM((NCH, K), jnp.int32),
        pltpu.VMEM((K, DEGW), jnp.float32),
        pltpu.VMEM((64, DEGW), jnp.float32),
        pltpu.VMEM_SHARED((NPAD, DEGW), jnp.float32),
        pltpu.SemaphoreType.DMA,
        pltpu.SemaphoreType.DMA,
    ],
)()


def _scatter_body(hs_hbm, src_hbm, dst_hbm, out_hbm, srcv, dstv, rows, zrows,
                  acc_sh):
    c = lax.axis_index("c")
    s = lax.axis_index("s")
    wid = c * NS + s

    def fill(i, _):
        def fill_row(k2, _2):
            zrows[i, pl.ds(k2 * 16, 16)] = jnp.zeros((16,), jnp.float32)
            return 0

        lax.fori_loop(0, C // 16, fill_row, 0)
        return 0

    lax.fori_loop(0, 16, fill, 0)

    def zero(t, _):
        pltpu.sync_copy(zrows, acc_sh.at[pl.ds(s * RPT + t * 16, 16)])
        return 0

    lax.fori_loop(0, RPT // 16, zero, 0)
    plsc.subcore_barrier()

    pltpu.sync_copy(src_hbm.at[wid], srcv)
    pltpu.sync_copy(dst_hbm.at[wid], dstv)

    def chunk(j, _):
        pltpu.sync_copy(hs_hbm.at[srcv.at[j]], rows)
        pltpu.sync_copy(rows, acc_sh.at[dstv.at[j]], add=True)
        return 0

    lax.fori_loop(0, NCH, chunk, 0)
    plsc.subcore_barrier()
    pltpu.sync_copy(acc_sh.at[pl.ds(s * RPT, RPT)],
                    out_hbm.at[c, pl.ds(s * RPT, RPT)])


_sc_scatter = functools.partial(
    pl.kernel,
    _scatter_body,
    out_type=jax.ShapeDtypeStruct((NC, NPAD, C), jnp.float32),
    mesh=_mesh,
    scratch_types=[
        pltpu.VMEM((NCH, K), jnp.int32),
        pltpu.VMEM((NCH, K), jnp.int32),
        pltpu.VMEM((K, C), jnp.float32),
        pltpu.VMEM((16, C), jnp.float32),
        pltpu.VMEM_SHARED((NPAD, C), jnp.float32),
    ],
)()


def _mm_body(x_ref, w_ref, h_ref):
    h_ref[...] = jnp.dot(x_ref[...], w_ref[...],
                         preferred_element_type=jnp.float32)


def _tc_mm(x, conv_w):
    return pl.pallas_call(
        _mm_body,
        grid=(N // _ROWS,),
        in_specs=[
            pl.BlockSpec((_ROWS, C), lambda i: (i, 0)),
            pl.BlockSpec((C, C), lambda i: (0, 0)),
        ],
        out_specs=pl.BlockSpec((_ROWS, C), lambda i: (i, 0)),
        out_shape=jax.ShapeDtypeStruct((N, C), jnp.float32),
    )(x, conv_w)


def _pre_body(h_ref, deg_ref, hs_ref, dinv_ref):
    db = deg_ref[...]
    d = db[0, :, 0:1] + db[1, :, 0:1] + 1.0
    dinv = 1.0 / jnp.sqrt(d)
    hs_ref[...] = h_ref[...] * dinv
    dinv_ref[...] = dinv


def _tc_pre(h, deg2):
    return pl.pallas_call(
        _pre_body,
        grid=(N // _ROWS,),
        in_specs=[
            pl.BlockSpec((_ROWS, C), lambda i: (i, 0)),
            pl.BlockSpec((NC, _ROWS, DEGW), lambda i: (0, i, 0)),
        ],
        out_specs=[
            pl.BlockSpec((_ROWS, C), lambda i: (i, 0)),
            pl.BlockSpec((_ROWS, 1), lambda i: (i, 0)),
        ],
        out_shape=[
            jax.ShapeDtypeStruct((N, C), jnp.float32),
            jax.ShapeDtypeStruct((N, 1), jnp.float32),
        ],
    )(h, deg2)


def _post_body(acc_ref, hs_ref, dinv_ref, x_ref, cb_ref, w1r, b1r, w2r, b2r,
               w3r, b3r, out):
    ab = acc_ref[...]
    pre = dinv_ref[...] * (ab[0] + ab[1] + hs_ref[...]) + cb_ref[...]
    o = jnp.maximum(pre, 0.0) + x_ref[...]
    h = jnp.dot(o, w1r[...], preferred_element_type=jnp.float32) + b1r[...]
    h = jnp.where(h >= 0, h, 0.01 * h)
    h = jnp.dot(h, w2r[...], preferred_element_type=jnp.float32) + b2r[...]
    h = jnp.where(h >= 0, h, 0.01 * h)
    a = jnp.dot(h, w3r[...], preferred_element_type=jnp.float32) + b3r[...]
    out[...] = jax.nn.softplus(a)


def _tc_post(acc2, hs, dinv, x, conv_b, w1, b1, w2, b2, w3, b3):
    full = lambda shape: pl.BlockSpec(shape, lambda i: tuple(0 for _ in shape))
    return pl.pallas_call(
        _post_body,
        grid=(N // _ROWS,),
        in_specs=[
            pl.BlockSpec((NC, _ROWS, C), lambda i: (0, i, 0)),
            pl.BlockSpec((_ROWS, C), lambda i: (i, 0)),
            pl.BlockSpec((_ROWS, 1), lambda i: (i, 0)),
            pl.BlockSpec((_ROWS, C), lambda i: (i, 0)),
            full((1, C)),
            full((C, H)),
            full((1, H)),
            full((H, H)),
            full((1, H)),
            full((H, 1)),
            full((1, 1)),
        ],
        out_specs=pl.BlockSpec((_ROWS, 1), lambda i: (i, 0)),
        out_shape=jax.ShapeDtypeStruct((N, 1), jnp.float32),
    )(acc2, hs, dinv, x, conv_b.reshape(1, C), w1, b1.reshape(1, H), w2,
      b2.reshape(1, H), w3, b3.reshape(1, 1))


def kernel(x, edge_index, conv_w, conv_b, w1, b1, w2, b2, w3, b3):
    src = edge_index[0]
    dst = edge_index[1]
    pad = EPAD - E
    srcp = jnp.concatenate([src, jnp.zeros((pad,), jnp.int32)])
    dstp = jnp.concatenate([dst, jnp.full((pad,), PAD_DST, jnp.int32)])
    srcp = srcp.reshape(NT, NCH, K)
    dstp = dstp.reshape(NT, NCH, K)

    deg_const = jnp.concatenate([jnp.ones((K, DEGW), jnp.float32),
                                 jnp.zeros((64, DEGW), jnp.float32)])
    h = _tc_mm(x, conv_w)
    deg2 = _sc_deg(dstp, deg_const)
    hs, dinv = _tc_pre(h, deg2)
    acc2 = _sc_scatter(hs, srcp, dstp)
    conc = _tc_post(acc2, hs, dinv, x, conv_b, w1, b1, w2, b2, w3, b3)

    alpha = conc.reshape(1, N) + 1e-20
    g = jax.random.gamma(jax.random.key(42), alpha)
    action = g / jnp.sum(g, axis=-1, keepdims=True)
    log_prob = (jnp.sum((alpha - 1.0) * jnp.log(action), axis=-1)
                + jax.lax.lgamma(jnp.sum(alpha, axis=-1))
                - jnp.sum(jax.lax.lgamma(alpha), axis=-1))
    action = jnp.squeeze(action, 0)[:, None]
    return (action, log_prob)


# speedup vs baseline: 1.0160x; 1.0160x over previous
import functools

import jax
import jax.numpy as jnp
from jax import lax
from jax.experimental import pallas as pl
from jax.experimental.pallas import tpu as pltpu
from jax.experimental.pallas import tpu_sc as plsc

N = 10000
E = 320000
C = 128
H = 128

NC = 2
NS = 16
NT = NC * NS
K = 128
NCH = 79
EPAD = NT * NCH * K
NPAD = 10240
PAD_DST = 10016
DEGW = 128
RPT = NPAD // NS
NBUF = 2
WCH = 16
NW = NCH // WCH
DDEPTH = 8

_ROWS = 1000

_mesh = plsc.VectorSubcoreMesh(core_axis_name="c", subcore_axis_name="s")


def _deg_body(dst_hbm, const_hbm, out_hbm, dstv, onesv, zv, deg_sh, zsem,
              dsem):
    c = lax.axis_index("c")
    s = lax.axis_index("s")
    wid = c * NS + s

    pltpu.sync_copy(const_hbm.at[pl.ds(0, K)], onesv)
    pltpu.sync_copy(const_hbm.at[pl.ds(K, 64)], zv)

    nz = RPT // 64
    for t in range(nz):
        pltpu.async_copy(zv, deg_sh.at[pl.ds(s * RPT + t * 64, 64)], zsem)
    pltpu.sync_copy(dst_hbm.at[wid], dstv)
    for t in range(nz):
        pltpu.make_async_copy(zv, deg_sh.at[pl.ds(s * RPT + t * 64, 64)],
                              zsem).wait()
    plsc.subcore_barrier()

    for b in range(DDEPTH):
        pltpu.async_copy(onesv, deg_sh.at[dstv.at[b]], dsem, add=True)

    def chunk(j, _):
        pltpu.make_async_copy(onesv, deg_sh.at[dstv.at[j]], dsem).wait()
        pltpu.async_copy(onesv, deg_sh.at[dstv.at[j + DDEPTH]], dsem,
                         add=True)
        return 0

    lax.fori_loop(0, NCH - DDEPTH, chunk, 0)

    def drain(j, _):
        pltpu.make_async_copy(onesv, deg_sh.at[dstv.at[j]], dsem).wait()
        return 0

    lax.fori_loop(0, DDEPTH, drain, 0)
    plsc.subcore_barrier()
    pltpu.sync_copy(deg_sh.at[pl.ds(s * RPT, RPT)],
                    out_hbm.at[c, pl.ds(s * RPT, RPT)])


_sc_deg = functools.partial(
    pl.kernel,
    _deg_body,
    out_type=jax.ShapeDtypeStruct((NC, NPAD, DEGW), jnp.float32),
    mesh=_mesh,
    scratch_types=[
        pltpu.VMEM((NCH, K), jnp.int32),
        pltpu.VMEM((K, DEGW), jnp.float32),
        pltpu.VMEM((64, DEGW), jnp.float32),
        pltpu.VMEM_SHARED((NPAD, DEGW), jnp.float32),
        pltpu.SemaphoreType.DMA,
        pltpu.SemaphoreType.DMA,
    ],
)()


def _scatter_body(hs_hbm, src_hbm, dst_hbm, out_hbm, srcv, dstv, rows, zrows,
                  acc_sh):
    c = lax.axis_index("c")
    s = lax.axis_index("s")
    wid = c * NS + s

    def fill(i, _):
        def fill_row(k2, _2):
            zrows[i, pl.ds(k2 * 16, 16)] = jnp.zeros((16,), jnp.float32)
            return 0

        lax.fori_loop(0, C // 16, fill_row, 0)
        return 0

    lax.fori_loop(0, 16, fill, 0)

    def zero(t, _):
        pltpu.sync_copy(zrows, acc_sh.at[pl.ds(s * RPT + t * 16, 16)])
        return 0

    lax.fori_loop(0, RPT // 16, zero, 0)
    plsc.subcore_barrier()

    pltpu.sync_copy(src_hbm.at[wid], srcv)
    pltpu.sync_copy(dst_hbm.at[wid], dstv)

    def chunk(j, _):
        pltpu.sync_copy(hs_hbm.at[srcv.at[j]], rows)
        pltpu.sync_copy(rows, acc_sh.at[dstv.at[j]], add=True)
        return 0

    lax.fori_loop(0, NCH, chunk, 0)
    plsc.subcore_barrier()
    pltpu.sync_copy(acc_sh.at[pl.ds(s * RPT, RPT)],
                    out_hbm.at[c, pl.ds(s * RPT, RPT)])


_sc_scatter = functools.partial(
    pl.kernel,
    _scatter_body,
    out_type=jax.ShapeDtypeStruct((NC, NPAD, C), jnp.float32),
    mesh=_mesh,
    scratch_types=[
        pltpu.VMEM((NCH, K), jnp.int32),
        pltpu.VMEM((NCH, K), jnp.int32),
        pltpu.VMEM((K, C), jnp.float32),
        pltpu.VMEM((16, C), jnp.float32),
        pltpu.VMEM_SHARED((NPAD, C), jnp.float32),
    ],
)()


def _pre_body(x_ref, w_ref, deg_ref, hs_ref, dinv_ref):
    db = deg_ref[...]
    d = db[0, :, 0:1] + db[1, :, 0:1] + 1.0
    dinv = 1.0 / jnp.sqrt(d)
    hs_ref[...] = jnp.dot(x_ref[...], w_ref[...],
                          preferred_element_type=jnp.float32) * dinv
    dinv_ref[...] = dinv


def _tc_pre(x, conv_w, deg2):
    return pl.pallas_call(
        _pre_body,
        grid=(N // _ROWS,),
        in_specs=[
            pl.BlockSpec((_ROWS, C), lambda i: (i, 0)),
            pl.BlockSpec((C, C), lambda i: (0, 0)),
            pl.BlockSpec((NC, _ROWS, DEGW), lambda i: (0, i, 0)),
        ],
        out_specs=[
            pl.BlockSpec((_ROWS, C), lambda i: (i, 0)),
            pl.BlockSpec((_ROWS, 1), lambda i: (i, 0)),
        ],
        out_shape=[
            jax.ShapeDtypeStruct((N, C), jnp.float32),
            jax.ShapeDtypeStruct((N, 1), jnp.float32),
        ],
    )(x, conv_w, deg2)


def _post_body(acc_ref, hs_ref, dinv_ref, x_ref, cb_ref, w1r, b1r, w2r, b2r,
               w3r, b3r, out):
    ab = acc_ref[...]
    pre = dinv_ref[...] * (ab[0] + ab[1] + hs_ref[...]) + cb_ref[...]
    o = jnp.maximum(pre, 0.0) + x_ref[...]
    h = jnp.dot(o, w1r[...], preferred_element_type=jnp.float32) + b1r[...]
    h = jnp.where(h >= 0, h, 0.01 * h)
    h = jnp.dot(h, w2r[...], preferred_element_type=jnp.float32) + b2r[...]
    h = jnp.where(h >= 0, h, 0.01 * h)
    a = jnp.dot(h, w3r[...], preferred_element_type=jnp.float32) + b3r[...]
    out[...] = jax.nn.softplus(a)


def _tc_post(acc2, hs, dinv, x, conv_b, w1, b1, w2, b2, w3, b3):
    full = lambda shape: pl.BlockSpec(shape, lambda i: tuple(0 for _ in shape))
    return pl.pallas_call(
        _post_body,
        grid=(N // _ROWS,),
        in_specs=[
            pl.BlockSpec((NC, _ROWS, C), lambda i: (0, i, 0)),
            pl.BlockSpec((_ROWS, C), lambda i: (i, 0)),
            pl.BlockSpec((_ROWS, 1), lambda i: (i, 0)),
            pl.BlockSpec((_ROWS, C), lambda i: (i, 0)),
            full((1, C)),
            full((C, H)),
            full((1, H)),
            full((H, H)),
            full((1, H)),
            full((H, 1)),
            full((1, 1)),
        ],
        out_specs=pl.BlockSpec((_ROWS, 1), lambda i: (i, 0)),
        out_shape=jax.ShapeDtypeStruct((N, 1), jnp.float32),
    )(acc2, hs, dinv, x, conv_b.reshape(1, C), w1, b1.reshape(1, H), w2,
      b2.reshape(1, H), w3, b3.reshape(1, 1))


def kernel(x, edge_index, conv_w, conv_b, w1, b1, w2, b2, w3, b3):
    src = edge_index[0]
    dst = edge_index[1]
    pad = EPAD - E
    srcp = jnp.concatenate([src, jnp.zeros((pad,), jnp.int32)])
    dstp = jnp.concatenate([dst, jnp.full((pad,), PAD_DST, jnp.int32)])
    srcp = srcp.reshape(NT, NCH, K)
    dstp = dstp.reshape(NT, NCH, K)

    deg_const = jnp.concatenate([jnp.ones((K, DEGW), jnp.float32),
                                 jnp.zeros((64, DEGW), jnp.float32)])
    deg2 = _sc_deg(dstp, deg_const)
    hs, dinv = _tc_pre(x, conv_w, deg2)
    acc2 = _sc_scatter(hs, srcp, dstp)
    conc = _tc_post(acc2, hs, dinv, x, conv_b, w1, b1, w2, b2, w3, b3)

    alpha = conc.reshape(1, N) + 1e-20
    g = jax.random.gamma(jax.random.key(42), alpha)
    action = g / jnp.sum(g, axis=-1, keepdims=True)
    log_prob = (jnp.sum((alpha - 1.0) * jnp.log(action), axis=-1)
                + jax.lax.lgamma(jnp.sum(alpha, axis=-1))
                - jnp.sum(jax.lax.lgamma(alpha), axis=-1))
    action = jnp.squeeze(action, 0)[:, None]
    return (action, log_prob)
